# Initial kernel scaffold; baseline (speedup 1.0000x reference)
import jax
import jax.numpy as jnp
from jax.experimental import pallas as pl


def kernel(x, edge_index, edge_index_neg, W1, b1, W2, b2, We1, be1, We2, be2):
    raise NotImplementedError("write your pallas kernel here")


# speedup vs baseline: 1.3990x; 1.3990x over previous
import functools

import jax
import jax.numpy as jnp
from jax import lax
from jax.experimental import pallas as pl
from jax.experimental.pallas import tpu as pltpu
from jax.experimental.pallas import tpu_sc as plsc

EMB = 128
N_EDGES = 320000
N_TOTAL = 2 * N_EDGES
NUM_WORKERS = 32
CHUNK = N_TOTAL // NUM_WORKERS
WIN = 80

TC_BLOCK = 512
TC_GRID = N_TOTAL // TC_BLOCK
TC_POS_BLOCKS = N_EDGES // TC_BLOCK


def _sc_gather_mul(src, dst, x):
    mesh = plsc.VectorSubcoreMesh(
        core_axis_name="c", subcore_axis_name="s", num_cores=2, num_subcores=16
    )

    @functools.partial(
        pl.kernel,
        out_type=jax.ShapeDtypeStruct((N_TOTAL, EMB), jnp.float32),
        mesh=mesh,
        scratch_types=[
            pltpu.VMEM((WIN,), jnp.int32),
            pltpu.VMEM((WIN,), jnp.int32),
            pltpu.VMEM((WIN, EMB), jnp.float32),
            pltpu.VMEM((WIN, EMB), jnp.float32),
            pltpu.SemaphoreType.DMA,
            pltpu.SemaphoreType.DMA,
        ],
    )
    def gather_mul(src_hbm, dst_hbm, x_hbm, out_hbm, idx_a, idx_b, rows_a,
                   rows_b, sem_a, sem_b):
        wid = lax.axis_index("s") * 2 + lax.axis_index("c")
        base = wid * CHUNK

        @pl.loop(0, CHUNK, step=WIN)
        def _(off):
            b0 = base + off
            pltpu.sync_copy(src_hbm.at[pl.ds(b0, WIN)], idx_a)
            pltpu.sync_copy(dst_hbm.at[pl.ds(b0, WIN)], idx_b)
            cp_a = pltpu.async_copy(x_hbm.at[idx_a], rows_a, sem_a)
            cp_b = pltpu.async_copy(x_hbm.at[idx_b], rows_b, sem_b)
            cp_a.wait()
            cp_b.wait()

            @pl.loop(0, WIN)
            def _(r):
                for c in range(EMB // 16):
                    sl = pl.ds(c * 16, 16)
                    rows_a[r, sl] = rows_a[r, sl] * rows_b[r, sl]

            pltpu.sync_copy(rows_a, out_hbm.at[pl.ds(b0, WIN)])

    return gather_mul(src, dst, x)


def _tc_mlp_kernel(em_ref, w1_ref, b1_ref, w2_ref, b2_ref, we1_ref, be1_ref,
                   we2_ref, be2_ref, attr_ref, logit_ref):
    i = pl.program_id(0)
    d = jnp.maximum(em_ref[...], 0.0)
    he = jnp.maximum(
        jnp.dot(d, we1_ref[...], preferred_element_type=jnp.float32)
        + be1_ref[...], 0.0)
    logit = (jnp.dot(he, we2_ref[...], preferred_element_type=jnp.float32)
             + be2_ref[...])
    logit_ref[...] = jax.nn.sigmoid(logit)

    @pl.when(i < TC_POS_BLOCKS)
    def _():
        h = jnp.maximum(
            jnp.dot(d, w1_ref[...], preferred_element_type=jnp.float32)
            + b1_ref[...], 0.0)
        attr_ref[...] = jax.nn.sigmoid(
            jnp.dot(h, w2_ref[...], preferred_element_type=jnp.float32)
            + b2_ref[...])


def _tc_mlp(em, W1, b1, W2, b2, We1, be1, We2, be2):
    full = lambda s: pl.BlockSpec(s, lambda i: (0, 0))
    attr, logit = pl.pallas_call(
        _tc_mlp_kernel,
        grid=(TC_GRID,),
        in_specs=[
            pl.BlockSpec((TC_BLOCK, EMB), lambda i: (i, 0)),
            full((EMB, EMB)),
            full((1, EMB)),
            full((EMB, 7)),
            full((1, 7)),
            full((EMB, EMB)),
            full((1, EMB)),
            full((EMB, 1)),
            full((1, 1)),
        ],
        out_specs=[
            pl.BlockSpec((TC_BLOCK, 7),
                         lambda i: (jnp.minimum(i, TC_POS_BLOCKS - 1), 0)),
            pl.BlockSpec((TC_BLOCK, 1), lambda i: (i, 0)),
        ],
        out_shape=[
            jax.ShapeDtypeStruct((N_EDGES, 7), jnp.float32),
            jax.ShapeDtypeStruct((N_TOTAL, 1), jnp.float32),
        ],
    )(em, W1, b1.reshape(1, EMB), W2, b2.reshape(1, 7), We1,
      be1.reshape(1, EMB), We2, be2.reshape(1, 1))
    return attr, logit


def kernel(x, edge_index, edge_index_neg, W1, b1, W2, b2, We1, be1, We2, be2):
    src = jnp.concatenate([edge_index[0], edge_index_neg[0]])
    dst = jnp.concatenate([edge_index[1], edge_index_neg[1]])
    em = _sc_gather_mul(src, dst, x)
    attr, logit = _tc_mlp(em, W1, b1, W2, b2, We1, be1, We2, be2)
    edge_pos = logit[:N_EDGES, 0]
    edge_neg = logit[N_EDGES:, 0]
    return (attr, edge_pos, edge_neg)


# speedup vs baseline: 4.1334x; 2.9546x over previous
import dataclasses
import functools

import jax
import jax.numpy as jnp
from jax import lax
from jax.experimental import pallas as pl
from jax.experimental.pallas import tpu as pltpu
from jax.experimental.pallas import tpu_sc as plsc

EMB = 128
EMBW = EMB // 2
N_EDGES = 320000
N_TOTAL = 2 * N_EDGES
NUM_WORKERS = 32
CHUNK = N_EDGES // 16
WIN = 80
NWIN = CHUNK // WIN
NPAIR = NWIN // 2

TC_BLOCK = 3200
TC_POS_BLOCKS = N_EDGES // TC_BLOCK


def _sc_gather_mul(ei_pos, ei_neg, xp):
    mesh = plsc.VectorSubcoreMesh(
        core_axis_name="c", subcore_axis_name="s", num_cores=2, num_subcores=16
    )

    cp = pltpu.CompilerParams(use_tc_tiling_on_sc=False)
    if "needs_layout_passes" in pltpu.CompilerParams.__dataclass_fields__:
        cp = dataclasses.replace(cp, needs_layout_passes=False)

    @functools.partial(
        pl.kernel,
        out_type=jax.ShapeDtypeStruct((N_TOTAL, EMBW), jnp.int32),
        mesh=mesh,
        compiler_params=cp,
        scratch_types=[
            pltpu.VMEM((CHUNK,), jnp.int32),
            pltpu.VMEM((CHUNK,), jnp.int32),
            pltpu.VMEM((2, WIN, EMBW), jnp.int32),
            pltpu.VMEM((2, WIN, EMBW), jnp.int32),
            pltpu.VMEM((2, WIN, EMBW), jnp.int32),
            pltpu.SemaphoreType.DMA((2,)),
            pltpu.SemaphoreType.DMA((2,)),
            pltpu.SemaphoreType.DMA((2,)),
        ],
    )
    def gather_mul(ei_pos_hbm, ei_neg_hbm, x_hbm, out_hbm, idx_s, idx_d,
                   rows_a, rows_b, rows_c, sem_a, sem_b, sem_o):
        wid = lax.axis_index("s") * 2 + lax.axis_index("c")
        lane = wid % 16
        base_e = lane * CHUNK

        def run(ei_hbm, out_base):
            pltpu.sync_copy(ei_hbm.at[pl.ds(base_e, CHUNK)], idx_s)
            pltpu.sync_copy(ei_hbm.at[pl.ds(N_EDGES + base_e, CHUNK)], idx_d)

            def start_gather(slot, w):
                off = w * WIN
                pltpu.async_copy(x_hbm.at[idx_s.at[pl.ds(off, WIN)]],
                                 rows_a.at[slot], sem_a.at[slot])
                pltpu.async_copy(x_hbm.at[idx_d.at[pl.ds(off, WIN)]],
                                 rows_b.at[slot], sem_b.at[slot])

            def wait_gather(slot):
                pltpu.make_async_copy(x_hbm.at[idx_s.at[pl.ds(0, WIN)]],
                                      rows_a.at[slot], sem_a.at[slot]).wait()
                pltpu.make_async_copy(x_hbm.at[idx_d.at[pl.ds(0, WIN)]],
                                      rows_b.at[slot], sem_b.at[slot]).wait()

            def wait_out(slot):
                pltpu.make_async_copy(
                    rows_c.at[slot], out_hbm.at[pl.ds(out_base, WIN)],
                    sem_o.at[slot]).wait()

            start_gather(0, 0)
            start_gather(1, 1)

            @pl.loop(0, NPAIR)
            def _(p):
                for slot in (0, 1):
                    w = 2 * p + slot
                    wait_gather(slot)

                    @pl.when(p > 0)
                    def _():
                        wait_out(slot)

                    @pl.loop(0, WIN, step=4)
                    def _(r):
                        for rr in range(4):
                            for c in range(EMBW // 16):
                                csl = pl.ds(c * 16, 16)
                                va = plsc.bitcast(rows_a[slot, r + rr, csl],
                                                  jnp.bfloat16)
                                vb = plsc.bitcast(rows_b[slot, r + rr, csl],
                                                  jnp.bfloat16)
                                prod = jnp.maximum(va * vb, jnp.bfloat16(0))
                                rows_c[slot, r + rr, csl] = plsc.bitcast(
                                    prod, jnp.int32)

                    pltpu.async_copy(
                        rows_c.at[slot],
                        out_hbm.at[pl.ds(out_base + w * WIN, WIN)],
                        sem_o.at[slot])

                    @pl.when(p < NPAIR - 1)
                    def _():
                        start_gather(slot, w + 2)

            wait_out(0)
            wait_out(1)

        @pl.when(wid < 16)
        def _():
            run(ei_pos_hbm, base_e)

        @pl.when(wid >= 16)
        def _():
            run(ei_neg_hbm, N_EDGES + base_e)

    return gather_mul(ei_pos.reshape(-1), ei_neg.reshape(-1), xp)


def _unpack(w):
    bf = jnp.bfloat16
    lo = lax.bitcast_convert_type(
        jnp.left_shift(w, 16), jnp.float32).astype(bf)
    hi = lax.bitcast_convert_type(w, jnp.float32).astype(bf)
    return jnp.concatenate([lo, hi], axis=1)


def _tc_mlp_kernel(em_p_ref, em_n_ref, w1_ref, b1_ref, w2_ref, b2_ref,
                   we1_ref, be1_ref, we2_ref, be2_ref,
                   attr_ref, lpos_ref, lneg_ref):
    hdims = (((0,), (1,)), ((), ()))
    bf = jnp.bfloat16
    d_p = _unpack(em_p_ref[...])
    d_n = _unpack(em_n_ref[...])
    he_p = jnp.maximum(
        jnp.dot(d_p, we1_ref[...],
                preferred_element_type=jnp.float32).astype(bf)
        + be1_ref[...], bf(0))
    he_n = jnp.maximum(
        jnp.dot(d_n, we1_ref[...],
                preferred_element_type=jnp.float32).astype(bf)
        + be1_ref[...], bf(0))
    h_p = jnp.maximum(
        jnp.dot(d_p, w1_ref[...],
                preferred_element_type=jnp.float32).astype(bf)
        + b1_ref[...], bf(0))
    lpos_ref[...] = jax.nn.sigmoid(
        lax.dot_general(we2_ref[...], he_p, hdims,
                        preferred_element_type=jnp.float32) + be2_ref[...])
    lneg_ref[...] = jax.nn.sigmoid(
        lax.dot_general(we2_ref[...], he_n, hdims,
                        preferred_element_type=jnp.float32) + be2_ref[...])
    attr_ref[...] = jax.nn.sigmoid(
        lax.dot_general(w2_ref[...], h_p, hdims,
                        preferred_element_type=jnp.float32) + b2_ref[...])


def _perm(w):
    return jnp.concatenate([w[0::2], w[1::2]], axis=0).astype(jnp.bfloat16)


def _tc_mlp(em, W1, b1, W2, b2, We1, be1, We2, be2):
    full = lambda s: pl.BlockSpec(s, lambda i: (0, 0))
    grid = N_EDGES // TC_BLOCK
    attr_t, lpos, lneg = pl.pallas_call(
        _tc_mlp_kernel,
        grid=(grid,),
        in_specs=[
            pl.BlockSpec((TC_BLOCK, EMBW), lambda i: (i, 0)),
            pl.BlockSpec((TC_BLOCK, EMBW), lambda i: (grid + i, 0)),
            full((EMB, EMB)),
            full((1, EMB)),
            full((EMB, 7)),
            full((7, 1)),
            full((EMB, EMB)),
            full((1, EMB)),
            full((EMB, 1)),
            full((1, 1)),
        ],
        out_specs=[
            pl.BlockSpec((7, TC_BLOCK), lambda i: (0, i)),
            pl.BlockSpec((1, TC_BLOCK), lambda i: (0, i)),
            pl.BlockSpec((1, TC_BLOCK), lambda i: (0, i)),
        ],
        out_shape=[
            jax.ShapeDtypeStruct((7, N_EDGES), jnp.float32),
            jax.ShapeDtypeStruct((1, N_EDGES), jnp.float32),
            jax.ShapeDtypeStruct((1, N_EDGES), jnp.float32),
        ],
    )(em, em, _perm(W1), b1.reshape(1, EMB).astype(jnp.bfloat16),
      W2.astype(jnp.bfloat16), b2.reshape(7, 1),
      _perm(We1), be1.reshape(1, EMB).astype(jnp.bfloat16),
      We2.astype(jnp.bfloat16), be2.reshape(1, 1))
    return attr_t, lpos, lneg


def kernel(x, edge_index, edge_index_neg, W1, b1, W2, b2, We1, be1, We2, be2):
    xb = x.astype(jnp.bfloat16)
    xp = lax.bitcast_convert_type(
        xb.reshape(x.shape[0], EMBW, 2), jnp.int32)
    em = _sc_gather_mul(edge_index, edge_index_neg, xp)
    attr_t, lpos, lneg = _tc_mlp(em, W1, b1, W2, b2, We1, be1, We2, be2)
    attr = attr_t.T
    edge_pos = lpos[0]
    edge_neg = lneg[0]
    return (attr, edge_pos, edge_neg)


# speedup vs baseline: 4.4547x; 1.0777x over previous
import dataclasses
import functools

import jax
import jax.numpy as jnp
from jax import lax
from jax.experimental import pallas as pl
from jax.experimental.pallas import tpu as pltpu
from jax.experimental.pallas import tpu_sc as plsc

EMB = 128
EMBW = EMB // 2
N_EDGES = 320000
N_TOTAL = 2 * N_EDGES
NUM_WORKERS = 32
CHUNK = N_EDGES // 16
WIN = 80
NWIN = CHUNK // WIN
NPAIR = NWIN // 2

TC_BLOCK = 2560
HB = TC_BLOCK // 2


def _sc_gather_mul(ei_pos, ei_neg, xp):
    mesh = plsc.VectorSubcoreMesh(
        core_axis_name="c", subcore_axis_name="s", num_cores=2, num_subcores=16
    )

    cp = pltpu.CompilerParams(use_tc_tiling_on_sc=False)
    if "needs_layout_passes" in pltpu.CompilerParams.__dataclass_fields__:
        cp = dataclasses.replace(cp, needs_layout_passes=False)

    @functools.partial(
        pl.kernel,
        out_type=jax.ShapeDtypeStruct((N_TOTAL // 2, EMB), jnp.int32),
        mesh=mesh,
        compiler_params=cp,
        scratch_types=[
            pltpu.VMEM((CHUNK,), jnp.int32),
            pltpu.VMEM((CHUNK,), jnp.int32),
            pltpu.VMEM((2, WIN, EMBW), jnp.int32),
            pltpu.VMEM((2, WIN, EMBW), jnp.int32),
            pltpu.VMEM((2, WIN // 2, EMB), jnp.int32),
            pltpu.SemaphoreType.DMA((2,)),
            pltpu.SemaphoreType.DMA((2,)),
            pltpu.SemaphoreType.DMA((2,)),
        ],
    )
    def gather_mul(ei_pos_hbm, ei_neg_hbm, x_hbm, out_hbm, idx_s, idx_d,
                   rows_a, rows_b, rows_c, sem_a, sem_b, sem_o):
        wid = lax.axis_index("s") * 2 + lax.axis_index("c")
        lane = wid % 16
        base_e = lane * CHUNK
        hwin = WIN // 2

        def run(ei_hbm, out_base):
            pltpu.sync_copy(ei_hbm.at[pl.ds(base_e, CHUNK)], idx_s)
            pltpu.sync_copy(ei_hbm.at[pl.ds(N_EDGES + base_e, CHUNK)], idx_d)

            def start_gather(slot, w):
                off = w * WIN
                pltpu.async_copy(x_hbm.at[idx_s.at[pl.ds(off, WIN)]],
                                 rows_a.at[slot], sem_a.at[slot])
                pltpu.async_copy(x_hbm.at[idx_d.at[pl.ds(off, WIN)]],
                                 rows_b.at[slot], sem_b.at[slot])

            def wait_gather(slot):
                pltpu.make_async_copy(x_hbm.at[idx_s.at[pl.ds(0, WIN)]],
                                      rows_a.at[slot], sem_a.at[slot]).wait()
                pltpu.make_async_copy(x_hbm.at[idx_d.at[pl.ds(0, WIN)]],
                                      rows_b.at[slot], sem_b.at[slot]).wait()

            def wait_out(slot):
                pltpu.make_async_copy(
                    rows_c.at[slot], out_hbm.at[pl.ds(out_base // 2, hwin)],
                    sem_o.at[slot]).wait()

            start_gather(0, 0)
            start_gather(1, 1)

            @pl.loop(0, NPAIR)
            def _(p):
                for slot in (0, 1):
                    w = 2 * p + slot
                    wait_gather(slot)

                    @pl.when(p > 0)
                    def _():
                        wait_out(slot)

                    @pl.loop(0, hwin, step=2)
                    def _(k):
                        for dr in (0, 1):
                            for half in (0, 1):
                                e = 2 * k + 2 * dr + half
                                for c in range(EMBW // 16):
                                    csl = pl.ds(c * 16, 16)
                                    va = plsc.bitcast(rows_a[slot, e, csl],
                                                      jnp.bfloat16)
                                    vb = plsc.bitcast(rows_b[slot, e, csl],
                                                      jnp.bfloat16)
                                    prod = jnp.maximum(va * vb,
                                                       jnp.bfloat16(0))
                                    osl = pl.ds(half * EMBW + c * 16, 16)
                                    rows_c[slot, k + dr, osl] = plsc.bitcast(
                                        prod, jnp.int32)

                    pltpu.async_copy(
                        rows_c.at[slot],
                        out_hbm.at[pl.ds(out_base // 2 + w * hwin, hwin)],
                        sem_o.at[slot])

                    @pl.when(p < NPAIR - 1)
                    def _():
                        start_gather(slot, w + 2)

            wait_out(0)
            wait_out(1)

        @pl.when(wid < 16)
        def _():
            run(ei_pos_hbm, base_e)

        @pl.when(wid >= 16)
        def _():
            run(ei_neg_hbm, N_EDGES + base_e)

    return gather_mul(ei_pos.reshape(-1), ei_neg.reshape(-1), xp)


def _unpack(w):
    bf = jnp.bfloat16
    lo = lax.bitcast_convert_type(
        jnp.left_shift(w, 16), jnp.float32).astype(bf)
    hi = lax.bitcast_convert_type(w, jnp.float32).astype(bf)
    return jnp.concatenate([lo, hi], axis=1)


def _tc_mlp_kernel(em_p_ref, em_n_ref, w1_ref, b1_ref, w2_ref, b2_ref,
                   we1_ref, be1_ref, we2_ref, be2_ref,
                   ae_ref, ao_ref, lpe_ref, lpo_ref, lne_ref, lno_ref):
    hdims = (((0,), (1,)), ((), ()))
    bf = jnp.bfloat16
    wp = em_p_ref[...]
    wn = em_n_ref[...]
    ds = [_unpack(wp[:, :EMBW]), _unpack(wp[:, EMBW:]),
          _unpack(wn[:, :EMBW]), _unpack(wn[:, EMBW:])]

    def hidden(d, w_ref, b_ref):
        return jnp.maximum(
            jnp.dot(d, w_ref[...],
                    preferred_element_type=jnp.float32).astype(bf)
            + b_ref[...], bf(0))

    def head(h, w_ref, b_ref):
        return jax.nn.sigmoid(
            lax.dot_general(w_ref[...], h, hdims,
                            preferred_element_type=jnp.float32) + b_ref[...])

    hes = [hidden(d, we1_ref, be1_ref) for d in ds]
    lpe_ref[...] = head(hes[0], we2_ref, be2_ref)
    lpo_ref[...] = head(hes[1], we2_ref, be2_ref)
    lne_ref[...] = head(hes[2], we2_ref, be2_ref)
    lno_ref[...] = head(hes[3], we2_ref, be2_ref)
    ae_ref[...] = head(hidden(ds[0], w1_ref, b1_ref), w2_ref, b2_ref)
    ao_ref[...] = head(hidden(ds[1], w1_ref, b1_ref), w2_ref, b2_ref)


def _perm(w):
    return jnp.concatenate([w[0::2], w[1::2]], axis=0).astype(jnp.bfloat16)


def _tc_mlp(em, W1, b1, W2, b2, We1, be1, We2, be2):
    full = lambda s: pl.BlockSpec(s, lambda i: (0, 0))
    grid = N_EDGES // TC_BLOCK
    half_edges = N_EDGES // 2
    narrow = lambda j: pl.BlockSpec((j, TC_BLOCK // 2), lambda i: (0, i))
    outs = pl.pallas_call(
        _tc_mlp_kernel,
        grid=(grid,),
        in_specs=[
            pl.BlockSpec((HB, EMB), lambda i: (i, 0)),
            pl.BlockSpec((HB, EMB), lambda i: (grid + i, 0)),
            full((EMB, EMB)),
            full((1, EMB)),
            full((EMB, 7)),
            full((7, 1)),
            full((EMB, EMB)),
            full((1, EMB)),
            full((EMB, 1)),
            full((1, 1)),
        ],
        out_specs=[narrow(7), narrow(7), narrow(1), narrow(1), narrow(1),
                   narrow(1)],
        out_shape=[
            jax.ShapeDtypeStruct((7, half_edges), jnp.float32),
            jax.ShapeDtypeStruct((7, half_edges), jnp.float32),
            jax.ShapeDtypeStruct((1, half_edges), jnp.float32),
            jax.ShapeDtypeStruct((1, half_edges), jnp.float32),
            jax.ShapeDtypeStruct((1, half_edges), jnp.float32),
            jax.ShapeDtypeStruct((1, half_edges), jnp.float32),
        ],
    )(em, em, _perm(W1), b1.reshape(1, EMB).astype(jnp.bfloat16),
      W2.astype(jnp.bfloat16), b2.reshape(7, 1),
      _perm(We1), be1.reshape(1, EMB).astype(jnp.bfloat16),
      We2.astype(jnp.bfloat16), be2.reshape(1, 1))
    return outs


def kernel(x, edge_index, edge_index_neg, W1, b1, W2, b2, We1, be1, We2, be2):
    xb = x.astype(jnp.bfloat16)
    xp = lax.bitcast_convert_type(
        xb.reshape(x.shape[0], EMBW, 2), jnp.int32)
    em = _sc_gather_mul(edge_index, edge_index_neg, xp)
    ae, ao, lpe, lpo, lne, lno = _tc_mlp(
        em, W1, b1, W2, b2, We1, be1, We2, be2)
    attr = jnp.stack([ae.T, ao.T], axis=1).reshape(N_EDGES, 7)
    edge_pos = jnp.stack([lpe[0], lpo[0]], axis=1).reshape(N_EDGES)
    edge_neg = jnp.stack([lne[0], lno[0]], axis=1).reshape(N_EDGES)
    return (attr, edge_pos, edge_neg)
